# baseline (speedup 1.0000x reference)
import functools

import jax
import jax.numpy as jnp
from jax import lax
from jax.experimental import pallas as pl
from jax.experimental.pallas import tpu as pltpu
from jax.experimental.pallas import tpu_sc as plsc

_NC = 2
_NS = 16
_NW = _NC * _NS
_CT = 60
_NCHUNK = 123
_TRASH = 16384


def _gather_body(ut_hbm, it_hbm, uidx_hbm, iidx_hbm, ue_hbm, ie_hbm,
                 idx_v, mmap, stage_a, stage_b, rmini, pmini, tk, tr,
                 sem_sa, sem_sb, sem_sc, sem_or, *, batch, ntiles):
    wid = lax.axis_index("s") * _NC + lax.axis_index("c")
    nt_lo = ntiles // _NW
    nextra = ntiles - nt_lo * _NW
    t0 = wid * nt_lo + jnp.minimum(wid, nextra)
    nt = nt_lo + jnp.where(wid < nextra, 1, 0)
    lo = t0 * 8
    nrows = nt * 8
    iota = lax.iota(jnp.int32, 16)
    nkb = batch // 16
    tkr = tk.at[:]
    trr = tr.at[:]
    mmr = mmap.at[:]
    nmb = (nt_lo + 1) * 8 // 16 + 1

    for tab, idxh, outh in ((ut_hbm, uidx_hbm, ue_hbm),
                            (it_hbm, iidx_hbm, ie_hbm)):
        pltpu.sync_copy(idxh, idx_v)

        def clear(q, carry):
            mmap[pl.ds(q * 16, 16)] = jnp.full((16,), -1, jnp.int32)
            return carry
        lax.fori_loop(0, nmb, clear, 0)

        def route(kb, carry):
            v = idx_v[pl.ds(kb * 16, 16)]
            rel = v - lo
            msk = (v >= lo) & (v < lo + nrows)
            plsc.store_scatter(mmr, [rel], kb * 16 + iota, mask=msk)
            return carry
        lax.fori_loop(0, nkb, route, 0)

        def flush(pos):
            pmini[...] = pos
            pltpu.async_copy(rmini, outh.at[pmini], sem_sc).wait()

        stages = (stage_a, stage_b)
        stsems = (sem_sa, sem_sb)

        def mk_per_hit(stg):
            def per_hit(j, carry):
                mbp, pos = carry
                k = plsc.load_gather(tkr, [jnp.full((16,), j, jnp.int32)])[0]
                r = plsc.load_gather(trr, [jnp.full((16,), j, jnp.int32)])[0]
                pltpu.sync_copy(stg.at[r >> 3, r & 7],
                                rmini.at[mbp, pl.ds(0, 64)])
                pos = jnp.where(iota == mbp, k, pos)

                @pl.when(mbp == 15)
                def _():
                    flush(pos)
                return ((mbp + 1) & 15, pos)
            return per_hit

        def mk_per_window(stg):
            per_hit = mk_per_hit(stg)

            def per_window(q, carry):
                mbp, pos, r0l = carry
                mv = mmap[pl.ds(r0l + q * 16, 16)]
                msk = mv >= 0
                cnt = plsc.all_reduce_population_count(msk)[0]

                plsc.store_compressed(tkr, mv, mask=msk)
                plsc.store_compressed(trr, q * 16 + iota, mask=msk)
                mbp, pos = lax.fori_loop(0, cnt, per_hit, (mbp, pos))
                return (mbp, pos, r0l)
            return per_window

        windows = (mk_per_window(stage_a), mk_per_window(stage_b))
        nch = (nt + _CT - 1) // _CT
        nch2 = ((nch + 1) // 2) * 2

        def cstart(c):
            return jnp.minimum(t0 + c * _CT, t0 + nt - _CT)

        def fire(c, par, tab=tab):
            pltpu.async_copy(tab.at[pl.ds(cstart(c), _CT)], stages[par],
                             stsems[par])

        fire(0, 0)
        fire(1, 1)

        def pair(cp, carry, tab=tab):
            mbp, pos = carry
            for par in (0, 1):
                c = cp * 2 + par
                pltpu.make_async_copy(tab.at[pl.ds(0, _CT)], stages[par],
                                      stsems[par]).wait()
                r0l = (cstart(c) - t0) * 8
                mbp, pos, _ = lax.fori_loop(0, (_CT * 8) // 16, windows[par],
                                            (mbp, pos, r0l))

                @pl.when(c + 2 < nch2)
                def _(c=c, par=par):
                    fire(c + 2, par)
            return (mbp, pos)

        mbp, pos = lax.fori_loop(0, nch2 // 2, pair, (jnp.int32(0), jnp.full(
            (16,), _TRASH, jnp.int32)))

        def per_orphan(j, carry, tab=tab, outh=outh):
            mbp, pos = carry
            k = plsc.load_gather(tkr, [jnp.full((16,), j, jnp.int32)])[0]
            row = plsc.load_gather(trr, [jnp.full((16,), j, jnp.int32)])[0]
            pltpu.async_copy(tab.at[row >> 3, row & 7],
                             rmini.at[mbp, pl.ds(0, 64)], sem_or).wait()
            pos = jnp.where(iota == mbp, k, pos)

            @pl.when(mbp == 15)
            def _():
                flush(pos)
            return ((mbp + 1) & 15, pos)

        def orphan_scan(kb, carry, tab=tab, outh=outh):
            v = idx_v[pl.ds(kb * 16, 16)]
            rel = v - lo
            msk = (v >= lo) & (v < lo + nrows)
            got = plsc.load_gather(mmr, [jnp.where(msk, rel, 0)])
            orph = msk & (got != kb * 16 + iota)
            cnt = plsc.all_reduce_population_count(orph)[0]
            plsc.store_compressed(tkr, kb * 16 + iota, mask=orph)
            plsc.store_compressed(trr, rel, mask=orph)
            return lax.fori_loop(0, cnt, per_orphan, carry)

        mbp, pos = lax.fori_loop(0, nkb, orphan_scan, (mbp, pos))

        @pl.when(mbp > 0)
        def _(mbp=mbp, pos=pos):
            flush(jnp.where(iota < mbp, pos, _TRASH))


def _sc_gather(user_table, item_table, uidx, iidx):
    batch = uidx.shape[0]
    nrows, dim = user_table.shape
    ntiles = nrows // 8
    out_rows = batch + 8
    mesh = plsc.VectorSubcoreMesh(core_axis_name="c", subcore_axis_name="s")
    ut3 = user_table.reshape(ntiles, 8, dim)
    it3 = item_table.reshape(ntiles, 8, dim)
    nt_max = ntiles // _NW + 1
    body = functools.partial(_gather_body, batch=batch, ntiles=ntiles)
    fn = pl.kernel(
        body,
        out_type=(jax.ShapeDtypeStruct((out_rows, 128), jnp.float32),
                  jax.ShapeDtypeStruct((out_rows, 128), jnp.float32)),
        mesh=mesh,
        scratch_types=[
            pltpu.VMEM((batch,), jnp.int32),
            pltpu.VMEM((nt_max * 8 + 16,), jnp.int32),
            pltpu.VMEM_SHARED((_CT, 8, dim), jnp.float32),
            pltpu.VMEM_SHARED((_CT, 8, dim), jnp.float32),
            pltpu.VMEM((16, 128), jnp.float32),
            pltpu.VMEM((16,), jnp.int32),
            pltpu.VMEM((16,), jnp.int32),
            pltpu.VMEM((16,), jnp.int32),
            pltpu.SemaphoreType.DMA,
            pltpu.SemaphoreType.DMA,
            pltpu.SemaphoreType.DMA,
            pltpu.SemaphoreType.DMA,
        ],
        compiler_params=pltpu.CompilerParams(needs_layout_passes=False),
    )
    return fn(ut3, it3, uidx.astype(jnp.int32), iidx.astype(jnp.int32))


def _mlp_body(ue_ref, ie_ref, w0u_ref, w0i_ref, b0_ref, w1_ref, b1_ref,
              w2_ref, b2_ref, wo_ref, bo_ref, out_ref):
    u = ue_ref[...][:, :64]
    v = ie_ref[...][:, :64]
    h = (jnp.dot(u, w0u_ref[...], preferred_element_type=jnp.float32)
         + jnp.dot(v, w0i_ref[...], preferred_element_type=jnp.float32)
         + b0_ref[...])
    h = jnp.maximum(h, 0.0)
    h = jnp.dot(h, w1_ref[...], preferred_element_type=jnp.float32) + b1_ref[...]
    h = jnp.maximum(h, 0.0)
    h = jnp.dot(h, w2_ref[...], preferred_element_type=jnp.float32) + b2_ref[...]
    h = jnp.maximum(h, 0.0)
    logits = jnp.sum(h * wo_ref[...], axis=1, keepdims=True) + bo_ref[...]
    out_ref[...] = jax.nn.sigmoid(logits)


def _mlp(ue, ie, W0, b0, W1, b1, W2, b2, Wo, bo, batch):
    dim = 64
    bt = 2048
    d0 = W0.shape[1]
    d1 = W1.shape[1]
    d2 = W2.shape[1]
    w0u = W0[:dim]
    w0i = W0[dim:]
    out = pl.pallas_call(
        _mlp_body,
        grid=(batch // bt,),
        in_specs=[
            pl.BlockSpec((bt, 128), lambda i: (i, 0)),
            pl.BlockSpec((bt, 128), lambda i: (i, 0)),
            pl.BlockSpec((dim, d0), lambda i: (0, 0)),
            pl.BlockSpec((dim, d0), lambda i: (0, 0)),
            pl.BlockSpec((1, d0), lambda i: (0, 0)),
            pl.BlockSpec((d0, d1), lambda i: (0, 0)),
            pl.BlockSpec((1, d1), lambda i: (0, 0)),
            pl.BlockSpec((d1, d2), lambda i: (0, 0)),
            pl.BlockSpec((1, d2), lambda i: (0, 0)),
            pl.BlockSpec((1, d2), lambda i: (0, 0)),
            pl.BlockSpec((1, 1), lambda i: (0, 0)),
        ],
        out_specs=pl.BlockSpec((bt, 1), lambda i: (i, 0)),
        out_shape=jax.ShapeDtypeStruct((batch, 1), jnp.float32),
    )(ue, ie, w0u, w0i, b0.reshape(1, d0), W1, b1.reshape(1, d1),
      W2, b2.reshape(1, d2), Wo.reshape(1, d2), bo.reshape(1, 1))
    return out[:, 0]


def kernel(user_indices, item_indices, user_table, item_table,
           W0, b0, W1, b1, W2, b2, Wo, bo):
    batch = user_indices.shape[0]
    ue, ie = _sc_gather(user_table, item_table, user_indices, item_indices)
    return _mlp(ue, ie, W0, b0, W1, b1, W2, b2, Wo, bo, batch)


# speedup vs baseline: 1.5982x; 1.5982x over previous
import functools

import jax
import jax.numpy as jnp
from jax import lax
from jax.experimental import pallas as pl
from jax.experimental.pallas import tpu as pltpu
from jax.experimental.pallas import tpu_sc as plsc

_NC = 2
_NS = 16
_NW = _NC * _NS


def _gather_body(ut_hbm, it_hbm, uidx_hbm, iidx_hbm, ue_hbm, ie_hbm,
                 idx_v, rows, sem, *, bpw):
    wid = lax.axis_index("s") * _NC + lax.axis_index("c")
    base = wid * bpw
    for tab, idxh, outh in ((ut_hbm, uidx_hbm, ue_hbm),
                            (it_hbm, iidx_hbm, ie_hbm)):
        pltpu.sync_copy(idxh.at[wid], idx_v)

        def issue(kb, carry, tab=tab):
            vu = idx_v[pl.ds(kb * 16, 16)]
            for l in range(16):
                pltpu.async_copy(tab.at[vu[l]], rows.at[kb * 16 + l], sem)
            return carry

        lax.fori_loop(0, bpw // 16, issue, 0)
        pltpu.make_async_copy(tab.at[pl.ds(0, bpw)], rows, sem).wait()
        pltpu.sync_copy(rows, outh.at[pl.ds(base, bpw)])


def _sc_gather(user_table, item_table, uidx, iidx):
    batch = uidx.shape[0]
    dim = user_table.shape[1]
    bpw = batch // _NW
    mesh = plsc.VectorSubcoreMesh(core_axis_name="c", subcore_axis_name="s")
    uidx2 = uidx.astype(jnp.int32).reshape(_NW, bpw)
    iidx2 = iidx.astype(jnp.int32).reshape(_NW, bpw)
    body = functools.partial(_gather_body, bpw=bpw)
    fn = pl.kernel(
        body,
        out_type=(jax.ShapeDtypeStruct((batch, dim), jnp.float32),
                  jax.ShapeDtypeStruct((batch, dim), jnp.float32)),
        mesh=mesh,
        scratch_types=[
            pltpu.VMEM((bpw,), jnp.int32),
            pltpu.VMEM((bpw, dim), jnp.float32),
            pltpu.SemaphoreType.DMA,
        ],
    )
    return fn(user_table, item_table, uidx2, iidx2)


def _mlp_body(ue_ref, ie_ref, w0u_ref, w0i_ref, b0_ref, w1_ref, b1_ref,
              w2_ref, b2_ref, wo_ref, bo_ref, out_ref):
    h = (jnp.dot(ue_ref[...], w0u_ref[...], preferred_element_type=jnp.float32)
         + jnp.dot(ie_ref[...], w0i_ref[...], preferred_element_type=jnp.float32)
         + b0_ref[...])
    h = jnp.maximum(h, 0.0)
    h = jnp.dot(h, w1_ref[...], preferred_element_type=jnp.float32) + b1_ref[...]
    h = jnp.maximum(h, 0.0)
    h = jnp.dot(h, w2_ref[...], preferred_element_type=jnp.float32) + b2_ref[...]
    h = jnp.maximum(h, 0.0)
    logits = jnp.sum(h * wo_ref[...], axis=1, keepdims=True) + bo_ref[...]
    out_ref[...] = jax.nn.sigmoid(logits)


def _mlp(ue, ie, W0, b0, W1, b1, W2, b2, Wo, bo):
    batch, dim = ue.shape
    bt = 2048
    d0 = W0.shape[1]
    d1 = W1.shape[1]
    d2 = W2.shape[1]
    w0u = W0[:dim]
    w0i = W0[dim:]
    out = pl.pallas_call(
        _mlp_body,
        grid=(batch // bt,),
        in_specs=[
            pl.BlockSpec((bt, dim), lambda i: (i, 0)),
            pl.BlockSpec((bt, dim), lambda i: (i, 0)),
            pl.BlockSpec((dim, d0), lambda i: (0, 0)),
            pl.BlockSpec((dim, d0), lambda i: (0, 0)),
            pl.BlockSpec((1, d0), lambda i: (0, 0)),
            pl.BlockSpec((d0, d1), lambda i: (0, 0)),
            pl.BlockSpec((1, d1), lambda i: (0, 0)),
            pl.BlockSpec((d1, d2), lambda i: (0, 0)),
            pl.BlockSpec((1, d2), lambda i: (0, 0)),
            pl.BlockSpec((1, d2), lambda i: (0, 0)),
            pl.BlockSpec((1, 1), lambda i: (0, 0)),
        ],
        out_specs=pl.BlockSpec((bt, 1), lambda i: (i, 0)),
        out_shape=jax.ShapeDtypeStruct((batch, 1), jnp.float32),
    )(ue, ie, w0u, w0i, b0.reshape(1, d0), W1, b1.reshape(1, d1),
      W2, b2.reshape(1, d2), Wo.reshape(1, d2), bo.reshape(1, 1))
    return out[:, 0]


def kernel(user_indices, item_indices, user_table, item_table,
           W0, b0, W1, b1, W2, b2, Wo, bo):
    ue, ie = _sc_gather(user_table, item_table, user_indices, item_indices)
    return _mlp(ue, ie, W0, b0, W1, b1, W2, b2, Wo, bo)
